# baseline (speedup 1.0000x reference)
import functools

import jax
import jax.numpy as jnp
from jax import lax
from jax.experimental import pallas as pl
from jax.experimental.pallas import tpu as pltpu
import jax.experimental.pallas.tpu_sc as plsc

N_ = 10000
M_ = 10000
NNZ_ = 320000
F = 64
EPS_ = 1e-5

NC = 2
NS = 16
NW = NC * NS
CH = 128
NCHUNK = 80
NNZP = NW * NCHUNK * CH
MT = M_ + 1
RPT = 624
REM = M_ - NS * RPT

_mesh = plsc.VectorSubcoreMesh(core_axis_name="c", subcore_axis_name="s")
_sc_params = pltpu.CompilerParams(use_tc_tiling_on_sc=False)


NBUF = 8
KPF = 5


def _make_scatter(with_counts):
  outs = jax.ShapeDtypeStruct((NC, M_, F), jnp.float32)
  if with_counts:
    outs = (outs,
            jax.ShapeDtypeStruct((NC, N_, 8), jnp.float32),
            jax.ShapeDtypeStruct((NC, M_, 8), jnp.float32))
  scratch = [
      pltpu.VMEM((NCHUNK, CH), jnp.int32),
      pltpu.VMEM((NCHUNK, CH), jnp.int32),
      pltpu.VMEM((NBUF, CH, F), jnp.float32),
      pltpu.VMEM_SHARED((MT, F), jnp.float32),
      [pltpu.SemaphoreType.DMA] * NBUF,
      [pltpu.SemaphoreType.DMA] * NBUF,
  ]
  if with_counts:
    scratch += [
        pltpu.VMEM((NCHUNK, CH), jnp.int32),
        pltpu.VMEM((CH, 8), jnp.float32),
        pltpu.VMEM_SHARED((N_ + 1, 8), jnp.float32),
        pltpu.VMEM_SHARED((M_ + 1, 8), jnp.float32),
        [pltpu.SemaphoreType.DMA] * NBUF,
        [pltpu.SemaphoreType.DMA] * NBUF,
    ]

  @functools.partial(
      pl.kernel,
      out_type=outs,
      mesh=_mesh,
      compiler_params=_sc_params,
      scratch_types=scratch,
  )
  def scatter_kernel(table_hbm, gidx_hbm, sidx_hbm, zeros_hbm, *rest):
    if with_counts:
      (cidx_hbm, ones_hbm, zer8_hbm, out_hbm, cs_out, ch_out,
       gidx_v, sidx_v, rows_v, acc_sh, gsems, ssems,
       cidx_v, ones_v, cs_sh, ch_sh, csems, hsems) = rest
    else:
      (out_hbm, gidx_v, sidx_v, rows_v, acc_sh, gsems, ssems) = rest
    c = lax.axis_index("c")
    s = lax.axis_index("s")
    wid = c * NS + s
    row0 = s * RPT
    pltpu.sync_copy(zeros_hbm.at[pl.ds(row0, RPT)], acc_sh.at[pl.ds(row0, RPT)])

    @pl.when(s == NS - 1)
    def _():
      pltpu.sync_copy(zeros_hbm.at[pl.ds(NS * RPT, REM)],
                      acc_sh.at[pl.ds(NS * RPT, REM)])

    pltpu.sync_copy(gidx_hbm.at[wid], gidx_v)
    pltpu.sync_copy(sidx_hbm.at[wid], sidx_v)
    if with_counts:
      pltpu.sync_copy(cidx_hbm.at[wid], cidx_v)
      pltpu.sync_copy(ones_hbm, ones_v)
      pltpu.sync_copy(zer8_hbm.at[pl.ds(row0, RPT)],
                      cs_sh.at[pl.ds(row0, RPT)])
      pltpu.sync_copy(zer8_hbm.at[pl.ds(row0, RPT)],
                      ch_sh.at[pl.ds(row0, RPT)])

      @pl.when(s == NS - 1)
      def _():
        pltpu.sync_copy(zer8_hbm.at[pl.ds(NS * RPT, REM)],
                        cs_sh.at[pl.ds(NS * RPT, REM)])
        pltpu.sync_copy(zer8_hbm.at[pl.ds(NS * RPT, REM)],
                        ch_sh.at[pl.ds(NS * RPT, REM)])

    plsc.subcore_barrier()

    def fire_gather(i, b):
      pltpu.async_copy(table_hbm.at[gidx_v.at[i]], rows_v.at[b], gsems[b])

    def wait_gather(b):
      pltpu.make_async_copy(table_hbm.at[gidx_v.at[0]], rows_v.at[b],
                            gsems[b]).wait()

    def fire_scatter(i, b):
      pltpu.async_copy(rows_v.at[b], acc_sh.at[sidx_v.at[i]], ssems[b],
                       add=True)

    def wait_scatter(b):
      pltpu.make_async_copy(rows_v.at[b], acc_sh.at[sidx_v.at[0]],
                            ssems[b]).wait()

    def fire_counts(i, b):
      pltpu.async_copy(ones_v, cs_sh.at[cidx_v.at[i]], csems[b], add=True)
      pltpu.async_copy(ones_v, ch_sh.at[sidx_v.at[i]], hsems[b], add=True)

    def drain_counts(b):
      pltpu.make_async_copy(ones_v, cs_sh.at[cidx_v.at[0]], csems[b]).wait()
      pltpu.make_async_copy(ones_v, ch_sh.at[sidx_v.at[0]], hsems[b]).wait()

    for b in range(KPF):
      fire_gather(b, b)

    def outer(i0, carry):
      for b in range(NBUF):
        i = i0 * NBUF + b
        wait_gather(b)
        fire_scatter(i, b)
        if with_counts:

          @pl.when(i >= NBUF)
          def _():
            drain_counts(b)

          fire_counts(i, b)
        pfs = (b + KPF) % NBUF

        @pl.when(i >= NBUF - KPF)
        def _():
          wait_scatter(pfs)

        @pl.when(i + KPF < NCHUNK)
        def _():
          fire_gather(i + KPF, pfs)

      return carry

    lax.fori_loop(0, NCHUNK // NBUF, outer, 0)
    for j in range(NBUF - KPF):
      wait_scatter((NCHUNK - 1 - j) % NBUF)
    if with_counts:
      for b in range(NBUF):
        drain_counts(b)
    plsc.subcore_barrier()
    pltpu.sync_copy(acc_sh.at[pl.ds(row0, RPT)], out_hbm.at[c, pl.ds(row0, RPT)])
    if with_counts:
      pltpu.sync_copy(cs_sh.at[pl.ds(row0, RPT)], cs_out.at[c, pl.ds(row0, RPT)])
      pltpu.sync_copy(ch_sh.at[pl.ds(row0, RPT)], ch_out.at[c, pl.ds(row0, RPT)])

    @pl.when(s == NS - 1)
    def _():
      pltpu.sync_copy(acc_sh.at[pl.ds(NS * RPT, REM)],
                      out_hbm.at[c, pl.ds(NS * RPT, REM)])
      if with_counts:
        pltpu.sync_copy(cs_sh.at[pl.ds(NS * RPT, REM)],
                        cs_out.at[c, pl.ds(NS * RPT, REM)])
        pltpu.sync_copy(ch_sh.at[pl.ds(NS * RPT, REM)],
                        ch_out.at[c, pl.ds(NS * RPT, REM)])

  return scatter_kernel


def _make_count():
  @functools.partial(
      pl.kernel,
      out_type=(jax.ShapeDtypeStruct((NC, N_, 8), jnp.float32),
                jax.ShapeDtypeStruct((NC, M_, 8), jnp.float32)),
      mesh=_mesh,
      compiler_params=_sc_params,
      scratch_types=[
          pltpu.VMEM((NCHUNK, CH), jnp.int32),
          pltpu.VMEM((NCHUNK, CH), jnp.int32),
          pltpu.VMEM((CH, 8), jnp.float32),
          pltpu.VMEM_SHARED((N_ + 1, 8), jnp.float32),
          pltpu.VMEM_SHARED((M_ + 1, 8), jnp.float32),
          [pltpu.SemaphoreType.DMA] * NBUF,
          [pltpu.SemaphoreType.DMA] * NBUF,
      ],
  )
  def count_kernel(src_hbm, he_hbm, ones_hbm, zer_hbm, cs_out, ch_out,
                   idxs_v, idxh_v, ones_v, cs_sh, ch_sh, ssems, hsems):
    c = lax.axis_index("c")
    s = lax.axis_index("s")
    wid = c * NS + s
    row0 = s * RPT
    pltpu.sync_copy(zer_hbm.at[pl.ds(row0, RPT)], cs_sh.at[pl.ds(row0, RPT)])
    pltpu.sync_copy(zer_hbm.at[pl.ds(row0, RPT)], ch_sh.at[pl.ds(row0, RPT)])

    @pl.when(s == NS - 1)
    def _():
      pltpu.sync_copy(zer_hbm.at[pl.ds(NS * RPT, REM)],
                      cs_sh.at[pl.ds(NS * RPT, REM)])
      pltpu.sync_copy(zer_hbm.at[pl.ds(NS * RPT, REM)],
                      ch_sh.at[pl.ds(NS * RPT, REM)])

    pltpu.sync_copy(ones_hbm, ones_v)
    pltpu.sync_copy(src_hbm.at[wid], idxs_v)
    pltpu.sync_copy(he_hbm.at[wid], idxh_v)
    plsc.subcore_barrier()

    def fire(i, b):
      pltpu.async_copy(ones_v, cs_sh.at[idxs_v.at[i]], ssems[b], add=True)
      pltpu.async_copy(ones_v, ch_sh.at[idxh_v.at[i]], hsems[b], add=True)

    def drain(b):
      pltpu.make_async_copy(ones_v, cs_sh.at[idxs_v.at[0]], ssems[b]).wait()
      pltpu.make_async_copy(ones_v, ch_sh.at[idxh_v.at[0]], hsems[b]).wait()

    def outer(i0, carry):
      for b in range(NBUF):
        i = i0 * NBUF + b

        @pl.when(i >= NBUF)
        def _():
          drain(b)

        fire(i, b)
      return carry

    lax.fori_loop(0, NCHUNK // NBUF, outer, 0)
    for b in range(NBUF):
      drain(b)
    plsc.subcore_barrier()
    pltpu.sync_copy(cs_sh.at[pl.ds(row0, RPT)], cs_out.at[c, pl.ds(row0, RPT)])
    pltpu.sync_copy(ch_sh.at[pl.ds(row0, RPT)], ch_out.at[c, pl.ds(row0, RPT)])

    @pl.when(s == NS - 1)
    def _():
      pltpu.sync_copy(cs_sh.at[pl.ds(NS * RPT, REM)],
                      cs_out.at[c, pl.ds(NS * RPT, REM)])
      pltpu.sync_copy(ch_sh.at[pl.ds(NS * RPT, REM)],
                      ch_out.at[c, pl.ds(NS * RPT, REM)])

  return count_kernel


_scatter = _make_scatter(False)
_count = _make_count()


_BR = 1000


def _mm0(x, w):
  def body(x_ref, w_ref, o_ref):
    o_ref[...] = jnp.dot(x_ref[...], w_ref[...],
                         preferred_element_type=jnp.float32)

  return pl.pallas_call(
      body,
      grid=(N_ // _BR,),
      in_specs=[pl.BlockSpec((_BR, 128), lambda i: (i, 0)),
                pl.BlockSpec((128, F), lambda i: (0, 0))],
      out_specs=pl.BlockSpec((_BR, F), lambda i: (i, 0)),
      out_shape=jax.ShapeDtypeStruct((N_, F), jnp.float32),
  )(x, w)


def _combine_e(parte, cnt_he):
  def body(p_ref, c_ref, o_ref):
    cnt = c_ref[0][:, 0:1] + c_ref[1][:, 0:1]
    binv = jnp.where(cnt > 0, 1.0 / cnt, 0.0)
    o_ref[...] = (p_ref[0] + p_ref[1]) * binv

  return pl.pallas_call(
      body,
      grid=(M_ // _BR,),
      in_specs=[pl.BlockSpec((NC, _BR, F), lambda i: (0, i, 0)),
                pl.BlockSpec((NC, _BR, 8), lambda i: (0, i, 0))],
      out_specs=pl.BlockSpec((_BR, F), lambda i: (i, 0)),
      out_shape=jax.ShapeDtypeStruct((M_, F), jnp.float32),
  )(parte, cnt_he)


def _dense(partn, cnt_src, scale, shift, w):
  def body(p_ref, c_ref, sc_ref, sh_ref, w_ref, o_ref):
    cnt = c_ref[0][:, 0:1] + c_ref[1][:, 0:1]
    dinv = jnp.where(cnt > 0, 1.0 / cnt, 0.0)
    h = (p_ref[0] + p_ref[1]) * dinv
    h = jnp.maximum(h * sc_ref[...] + sh_ref[...], 0.0)
    o_ref[...] = jnp.dot(h, w_ref[...], preferred_element_type=jnp.float32)

  return pl.pallas_call(
      body,
      grid=(N_ // _BR,),
      in_specs=[pl.BlockSpec((NC, _BR, F), lambda i: (0, i, 0)),
                pl.BlockSpec((NC, _BR, 8), lambda i: (0, i, 0)),
                pl.BlockSpec((1, F), lambda i: (0, 0)),
                pl.BlockSpec((1, F), lambda i: (0, 0)),
                pl.BlockSpec((F, F), lambda i: (0, 0))],
      out_specs=pl.BlockSpec((_BR, F), lambda i: (i, 0)),
      out_shape=jax.ShapeDtypeStruct((N_, F), jnp.float32),
  )(partn, cnt_src, scale, shift, w)


def _final(partn, cnt_src, scale, shift, wc1, bc1, wc2, bc2):
  def body(p_ref, c_ref, sc_ref, sh_ref, w1_ref, b1_ref, w2_ref, b2_ref,
           o_ref):
    cnt = c_ref[0][:, 0:1] + c_ref[1][:, 0:1]
    dinv = jnp.where(cnt > 0, 1.0 / cnt, 0.0)
    h = (p_ref[0] + p_ref[1]) * dinv
    h = jnp.maximum(h * sc_ref[...] + sh_ref[...], 0.0)
    t = jnp.dot(h, w1_ref[...], preferred_element_type=jnp.float32)
    t = jnp.maximum(t + b1_ref[...], 0.0)
    o_ref[...] = jnp.dot(t, w2_ref[...],
                         preferred_element_type=jnp.float32) + b2_ref[...]

  H2 = F // 2
  NCLS = 10
  return pl.pallas_call(
      body,
      grid=(N_ // _BR,),
      in_specs=[pl.BlockSpec((NC, _BR, F), lambda i: (0, i, 0)),
                pl.BlockSpec((NC, _BR, 8), lambda i: (0, i, 0)),
                pl.BlockSpec((1, F), lambda i: (0, 0)),
                pl.BlockSpec((1, F), lambda i: (0, 0)),
                pl.BlockSpec((F, H2), lambda i: (0, 0)),
                pl.BlockSpec((1, H2), lambda i: (0, 0)),
                pl.BlockSpec((H2, NCLS), lambda i: (0, 0)),
                pl.BlockSpec((1, NCLS), lambda i: (0, 0))],
      out_specs=pl.BlockSpec((_BR, NCLS), lambda i: (i, 0)),
      out_shape=jax.ShapeDtypeStruct((N_, NCLS), jnp.float32),
  )(partn, cnt_src, scale, shift, wc1, bc1, wc2, bc2)


def _affine(b, g, be, rm, rv):
  sc = g * lax.rsqrt(rv + EPS_)
  sh = (b - rm) * sc + be
  return sc.reshape(1, F), sh.reshape(1, F)


def kernel(x, hyperedge_index, W0, b0, g0, be0, rm0, rv0,
           W1, b1, g1, be1, rm1, rv1, W2, b2, g2, be2, rm2, rv2,
           Wc1, bc1, Wc2, bc2):
  src = hyperedge_index[0]
  he = hyperedge_index[1]
  pad = NNZP - NNZ_
  shp = (NW, NCHUNK, CH)
  zpad = jnp.zeros((pad,), jnp.int32)
  src_g = jnp.concatenate([src, zpad]).reshape(shp)
  he_g = jnp.concatenate([he, zpad]).reshape(shp)
  src_t = jnp.concatenate([src, jnp.full((pad,), N_, jnp.int32)]).reshape(shp)
  he_t = jnp.concatenate([he, jnp.full((pad,), M_, jnp.int32)]).reshape(shp)

  zeros_mf = jnp.zeros((M_, F), jnp.float32)
  zeros_8 = jnp.zeros((M_, 8), jnp.float32)
  ones_8 = jnp.ones((CH, 8), jnp.float32)

  affines = [_affine(b0, g0, be0, rm0, rv0),
             _affine(b1, g1, be1, rm1, rv1),
             _affine(b2, g2, be2, rm2, rv2)]
  ws_next = [W1, W2]

  cnt_src, cnt_he = _count(src_t, he_t, ones_8, zeros_8)
  xw = _mm0(x, W0)
  for l in range(3):
    parte = _scatter(xw, src_g, he_t, zeros_mf)
    oute = _combine_e(parte, cnt_he)
    partn = _scatter(oute, he_g, src_t, zeros_mf)
    sc, sh = affines[l]
    if l < 2:
      xw = _dense(partn, cnt_src, sc, sh, ws_next[l])
    else:
      out = _final(partn, cnt_src, sc, sh, Wc1, bc1.reshape(1, -1),
                   Wc2, bc2.reshape(1, -1))
  return out


# speedup vs baseline: 1.2152x; 1.2152x over previous
import functools

import jax
import jax.numpy as jnp
from jax import lax
from jax.experimental import pallas as pl
from jax.experimental.pallas import tpu as pltpu
import jax.experimental.pallas.tpu_sc as plsc

N_ = 10000
M_ = 10000
NNZ_ = 320000
F = 64
EPS_ = 1e-5

NC = 2
NS = 16
NW = NC * NS
CH = 128
NCHUNK = 80
NNZP = NW * NCHUNK * CH
NDUM = 128
MT = M_ + NDUM
RPT = 624
REM = M_ - NS * RPT

_mesh = plsc.VectorSubcoreMesh(core_axis_name="c", subcore_axis_name="s")
_sc_params = pltpu.CompilerParams(use_tc_tiling_on_sc=False)


NBUF = 8
KPF = 5


def _make_scatter(with_counts):
  outs = jax.ShapeDtypeStruct((NC, M_, F), jnp.float32)
  if with_counts:
    outs = (outs,
            jax.ShapeDtypeStruct((NC, N_, 8), jnp.float32),
            jax.ShapeDtypeStruct((NC, M_, 8), jnp.float32))
  scratch = [
      pltpu.VMEM((NCHUNK, CH), jnp.int32),
      pltpu.VMEM((NCHUNK, CH), jnp.int32),
      pltpu.VMEM((NBUF, CH, F), jnp.float32),
      pltpu.VMEM_SHARED((MT, F), jnp.float32),
      [pltpu.SemaphoreType.DMA] * NBUF,
      [pltpu.SemaphoreType.DMA] * NBUF,
  ]
  if with_counts:
    scratch += [
        pltpu.VMEM((NCHUNK, CH), jnp.int32),
        pltpu.VMEM((CH, 8), jnp.float32),
        pltpu.VMEM_SHARED((N_ + 1, 8), jnp.float32),
        pltpu.VMEM_SHARED((M_ + 1, 8), jnp.float32),
        [pltpu.SemaphoreType.DMA] * NBUF,
        [pltpu.SemaphoreType.DMA] * NBUF,
    ]

  @functools.partial(
      pl.kernel,
      out_type=outs,
      mesh=_mesh,
      compiler_params=_sc_params,
      scratch_types=scratch,
  )
  def scatter_kernel(table_hbm, gidx_hbm, sidx_hbm, zeros_hbm, *rest):
    if with_counts:
      (cidx_hbm, ones_hbm, zer8_hbm, out_hbm, cs_out, ch_out,
       gidx_v, sidx_v, rows_v, acc_sh, gsems, ssems,
       cidx_v, ones_v, cs_sh, ch_sh, csems, hsems) = rest
    else:
      (out_hbm, gidx_v, sidx_v, rows_v, acc_sh, gsems, ssems) = rest
    c = lax.axis_index("c")
    s = lax.axis_index("s")
    wid = c * NS + s
    row0 = s * RPT
    pltpu.sync_copy(zeros_hbm.at[pl.ds(row0, RPT)], acc_sh.at[pl.ds(row0, RPT)])

    @pl.when(s == NS - 1)
    def _():
      pltpu.sync_copy(zeros_hbm.at[pl.ds(NS * RPT, REM)],
                      acc_sh.at[pl.ds(NS * RPT, REM)])

    pltpu.sync_copy(gidx_hbm.at[wid], gidx_v)
    pltpu.sync_copy(sidx_hbm.at[wid], sidx_v)
    if with_counts:
      pltpu.sync_copy(cidx_hbm.at[wid], cidx_v)
      pltpu.sync_copy(ones_hbm, ones_v)
      pltpu.sync_copy(zer8_hbm.at[pl.ds(row0, RPT)],
                      cs_sh.at[pl.ds(row0, RPT)])
      pltpu.sync_copy(zer8_hbm.at[pl.ds(row0, RPT)],
                      ch_sh.at[pl.ds(row0, RPT)])

      @pl.when(s == NS - 1)
      def _():
        pltpu.sync_copy(zer8_hbm.at[pl.ds(NS * RPT, REM)],
                        cs_sh.at[pl.ds(NS * RPT, REM)])
        pltpu.sync_copy(zer8_hbm.at[pl.ds(NS * RPT, REM)],
                        ch_sh.at[pl.ds(NS * RPT, REM)])

    plsc.subcore_barrier()

    def fire_gather(i, b):
      pltpu.async_copy(table_hbm.at[gidx_v.at[i]], rows_v.at[b], gsems[b])

    def wait_gather(b):
      pltpu.make_async_copy(table_hbm.at[gidx_v.at[0]], rows_v.at[b],
                            gsems[b]).wait()

    def fire_scatter(i, b):
      pltpu.async_copy(rows_v.at[b], acc_sh.at[sidx_v.at[i]], ssems[b],
                       add=True)

    def wait_scatter(b):
      pltpu.make_async_copy(rows_v.at[b], acc_sh.at[sidx_v.at[0]],
                            ssems[b]).wait()

    def fire_counts(i, b):
      pltpu.async_copy(ones_v, cs_sh.at[cidx_v.at[i]], csems[b], add=True)
      pltpu.async_copy(ones_v, ch_sh.at[sidx_v.at[i]], hsems[b], add=True)

    def drain_counts(b):
      pltpu.make_async_copy(ones_v, cs_sh.at[cidx_v.at[0]], csems[b]).wait()
      pltpu.make_async_copy(ones_v, ch_sh.at[sidx_v.at[0]], hsems[b]).wait()

    for b in range(KPF):
      fire_gather(b, b)

    def outer(i0, carry):
      for b in range(NBUF):
        i = i0 * NBUF + b
        wait_gather(b)
        fire_scatter(i, b)
        if with_counts:

          @pl.when(i >= NBUF)
          def _():
            drain_counts(b)

          fire_counts(i, b)
        pfs = (b + KPF) % NBUF

        @pl.when(i >= NBUF - KPF)
        def _():
          wait_scatter(pfs)

        @pl.when(i + KPF < NCHUNK)
        def _():
          fire_gather(i + KPF, pfs)

      return carry

    lax.fori_loop(0, NCHUNK // NBUF, outer, 0)
    for j in range(NBUF - KPF):
      wait_scatter((NCHUNK - 1 - j) % NBUF)
    if with_counts:
      for b in range(NBUF):
        drain_counts(b)
    plsc.subcore_barrier()
    pltpu.sync_copy(acc_sh.at[pl.ds(row0, RPT)], out_hbm.at[c, pl.ds(row0, RPT)])
    if with_counts:
      pltpu.sync_copy(cs_sh.at[pl.ds(row0, RPT)], cs_out.at[c, pl.ds(row0, RPT)])
      pltpu.sync_copy(ch_sh.at[pl.ds(row0, RPT)], ch_out.at[c, pl.ds(row0, RPT)])

    @pl.when(s == NS - 1)
    def _():
      pltpu.sync_copy(acc_sh.at[pl.ds(NS * RPT, REM)],
                      out_hbm.at[c, pl.ds(NS * RPT, REM)])
      if with_counts:
        pltpu.sync_copy(cs_sh.at[pl.ds(NS * RPT, REM)],
                        cs_out.at[c, pl.ds(NS * RPT, REM)])
        pltpu.sync_copy(ch_sh.at[pl.ds(NS * RPT, REM)],
                        ch_out.at[c, pl.ds(NS * RPT, REM)])

  return scatter_kernel


def _make_count():
  @functools.partial(
      pl.kernel,
      out_type=(jax.ShapeDtypeStruct((NC, N_, 8), jnp.float32),
                jax.ShapeDtypeStruct((NC, M_, 8), jnp.float32)),
      mesh=_mesh,
      compiler_params=_sc_params,
      scratch_types=[
          pltpu.VMEM((NCHUNK, CH), jnp.int32),
          pltpu.VMEM((NCHUNK, CH), jnp.int32),
          pltpu.VMEM((CH, 8), jnp.float32),
          pltpu.VMEM_SHARED((N_ + NDUM, 8), jnp.float32),
          pltpu.VMEM_SHARED((M_ + NDUM, 8), jnp.float32),
          [pltpu.SemaphoreType.DMA] * NBUF,
          [pltpu.SemaphoreType.DMA] * NBUF,
      ],
  )
  def count_kernel(src_hbm, he_hbm, ones_hbm, zer_hbm, cs_out, ch_out,
                   idxs_v, idxh_v, ones_v, cs_sh, ch_sh, ssems, hsems):
    c = lax.axis_index("c")
    s = lax.axis_index("s")
    wid = c * NS + s
    row0 = s * RPT
    pltpu.sync_copy(zer_hbm.at[pl.ds(row0, RPT)], cs_sh.at[pl.ds(row0, RPT)])
    pltpu.sync_copy(zer_hbm.at[pl.ds(row0, RPT)], ch_sh.at[pl.ds(row0, RPT)])

    @pl.when(s == NS - 1)
    def _():
      pltpu.sync_copy(zer_hbm.at[pl.ds(NS * RPT, REM)],
                      cs_sh.at[pl.ds(NS * RPT, REM)])
      pltpu.sync_copy(zer_hbm.at[pl.ds(NS * RPT, REM)],
                      ch_sh.at[pl.ds(NS * RPT, REM)])

    pltpu.sync_copy(ones_hbm, ones_v)
    pltpu.sync_copy(src_hbm.at[wid], idxs_v)
    pltpu.sync_copy(he_hbm.at[wid], idxh_v)
    plsc.subcore_barrier()

    def fire(i, b):
      pltpu.async_copy(ones_v, cs_sh.at[idxs_v.at[i]], ssems[b], add=True)
      pltpu.async_copy(ones_v, ch_sh.at[idxh_v.at[i]], hsems[b], add=True)

    def drain(b):
      pltpu.make_async_copy(ones_v, cs_sh.at[idxs_v.at[0]], ssems[b]).wait()
      pltpu.make_async_copy(ones_v, ch_sh.at[idxh_v.at[0]], hsems[b]).wait()

    def outer(i0, carry):
      for b in range(NBUF):
        i = i0 * NBUF + b

        @pl.when(i >= NBUF)
        def _():
          drain(b)

        fire(i, b)
      return carry

    lax.fori_loop(0, NCHUNK // NBUF, outer, 0)
    for b in range(NBUF):
      drain(b)
    plsc.subcore_barrier()
    pltpu.sync_copy(cs_sh.at[pl.ds(row0, RPT)], cs_out.at[c, pl.ds(row0, RPT)])
    pltpu.sync_copy(ch_sh.at[pl.ds(row0, RPT)], ch_out.at[c, pl.ds(row0, RPT)])

    @pl.when(s == NS - 1)
    def _():
      pltpu.sync_copy(cs_sh.at[pl.ds(NS * RPT, REM)],
                      cs_out.at[c, pl.ds(NS * RPT, REM)])
      pltpu.sync_copy(ch_sh.at[pl.ds(NS * RPT, REM)],
                      ch_out.at[c, pl.ds(NS * RPT, REM)])

  return count_kernel


_scatter = _make_scatter(False)
_count = _make_count()


_BR = 1000


def _mm0(x, w):
  def body(x_ref, w_ref, o_ref):
    o_ref[...] = jnp.dot(x_ref[...], w_ref[...],
                         preferred_element_type=jnp.float32)

  return pl.pallas_call(
      body,
      grid=(N_ // _BR,),
      in_specs=[pl.BlockSpec((_BR, 128), lambda i: (i, 0)),
                pl.BlockSpec((128, F), lambda i: (0, 0))],
      out_specs=pl.BlockSpec((_BR, F), lambda i: (i, 0)),
      out_shape=jax.ShapeDtypeStruct((N_, F), jnp.float32),
  )(x, w)


def _combine_e(parte, cnt_he):
  def body(p_ref, c_ref, o_ref):
    cnt = c_ref[0][:, 0:1] + c_ref[1][:, 0:1]
    binv = jnp.where(cnt > 0, 1.0 / cnt, 0.0)
    o_ref[...] = (p_ref[0] + p_ref[1]) * binv

  return pl.pallas_call(
      body,
      grid=(M_ // _BR,),
      in_specs=[pl.BlockSpec((NC, _BR, F), lambda i: (0, i, 0)),
                pl.BlockSpec((NC, _BR, 8), lambda i: (0, i, 0))],
      out_specs=pl.BlockSpec((_BR, F), lambda i: (i, 0)),
      out_shape=jax.ShapeDtypeStruct((M_, F), jnp.float32),
  )(parte, cnt_he)


def _dense(partn, cnt_src, scale, shift, w):
  def body(p_ref, c_ref, sc_ref, sh_ref, w_ref, o_ref):
    cnt = c_ref[0][:, 0:1] + c_ref[1][:, 0:1]
    dinv = jnp.where(cnt > 0, 1.0 / cnt, 0.0)
    h = (p_ref[0] + p_ref[1]) * dinv
    h = jnp.maximum(h * sc_ref[...] + sh_ref[...], 0.0)
    o_ref[...] = jnp.dot(h, w_ref[...], preferred_element_type=jnp.float32)

  return pl.pallas_call(
      body,
      grid=(N_ // _BR,),
      in_specs=[pl.BlockSpec((NC, _BR, F), lambda i: (0, i, 0)),
                pl.BlockSpec((NC, _BR, 8), lambda i: (0, i, 0)),
                pl.BlockSpec((1, F), lambda i: (0, 0)),
                pl.BlockSpec((1, F), lambda i: (0, 0)),
                pl.BlockSpec((F, F), lambda i: (0, 0))],
      out_specs=pl.BlockSpec((_BR, F), lambda i: (i, 0)),
      out_shape=jax.ShapeDtypeStruct((N_, F), jnp.float32),
  )(partn, cnt_src, scale, shift, w)


def _final(partn, cnt_src, scale, shift, wc1, bc1, wc2, bc2):
  def body(p_ref, c_ref, sc_ref, sh_ref, w1_ref, b1_ref, w2_ref, b2_ref,
           o_ref):
    cnt = c_ref[0][:, 0:1] + c_ref[1][:, 0:1]
    dinv = jnp.where(cnt > 0, 1.0 / cnt, 0.0)
    h = (p_ref[0] + p_ref[1]) * dinv
    h = jnp.maximum(h * sc_ref[...] + sh_ref[...], 0.0)
    t = jnp.dot(h, w1_ref[...], preferred_element_type=jnp.float32)
    t = jnp.maximum(t + b1_ref[...], 0.0)
    o_ref[...] = jnp.dot(t, w2_ref[...],
                         preferred_element_type=jnp.float32) + b2_ref[...]

  H2 = F // 2
  NCLS = 10
  return pl.pallas_call(
      body,
      grid=(N_ // _BR,),
      in_specs=[pl.BlockSpec((NC, _BR, F), lambda i: (0, i, 0)),
                pl.BlockSpec((NC, _BR, 8), lambda i: (0, i, 0)),
                pl.BlockSpec((1, F), lambda i: (0, 0)),
                pl.BlockSpec((1, F), lambda i: (0, 0)),
                pl.BlockSpec((F, H2), lambda i: (0, 0)),
                pl.BlockSpec((1, H2), lambda i: (0, 0)),
                pl.BlockSpec((H2, NCLS), lambda i: (0, 0)),
                pl.BlockSpec((1, NCLS), lambda i: (0, 0))],
      out_specs=pl.BlockSpec((_BR, NCLS), lambda i: (i, 0)),
      out_shape=jax.ShapeDtypeStruct((N_, NCLS), jnp.float32),
  )(partn, cnt_src, scale, shift, wc1, bc1, wc2, bc2)


def _affine(b, g, be, rm, rv):
  sc = g * lax.rsqrt(rv + EPS_)
  sh = (b - rm) * sc + be
  return sc.reshape(1, F), sh.reshape(1, F)


def kernel(x, hyperedge_index, W0, b0, g0, be0, rm0, rv0,
           W1, b1, g1, be1, rm1, rv1, W2, b2, g2, be2, rm2, rv2,
           Wc1, bc1, Wc2, bc2):
  per_tile = NNZ_ // NW
  padw = NCHUNK * CH - per_tile
  shp = (NW, NCHUNK, CH)
  src2 = hyperedge_index[0].reshape(NW, per_tile)
  he2 = hyperedge_index[1].reshape(NW, per_tile)
  gpad = jnp.zeros((NW, padw), jnp.int32)
  tpad = jnp.broadcast_to(jnp.arange(padw, dtype=jnp.int32) % NDUM,
                          (NW, padw))
  src_g = jnp.concatenate([src2, gpad], axis=1).reshape(shp)
  he_g = jnp.concatenate([he2, gpad], axis=1).reshape(shp)
  src_t = jnp.concatenate([src2, N_ + tpad], axis=1).reshape(shp)
  he_t = jnp.concatenate([he2, M_ + tpad], axis=1).reshape(shp)

  zeros_mf = jnp.zeros((M_, F), jnp.float32)
  zeros_8 = jnp.zeros((M_, 8), jnp.float32)
  ones_8 = jnp.ones((CH, 8), jnp.float32)

  affines = [_affine(b0, g0, be0, rm0, rv0),
             _affine(b1, g1, be1, rm1, rv1),
             _affine(b2, g2, be2, rm2, rv2)]
  ws_next = [W1, W2]

  cnt_src, cnt_he = _count(src_t, he_t, ones_8, zeros_8)
  xw = _mm0(x, W0)
  for l in range(3):
    parte = _scatter(xw, src_g, he_t, zeros_mf)
    oute = _combine_e(parte, cnt_he)
    partn = _scatter(oute, he_g, src_t, zeros_mf)
    sc, sh = affines[l]
    if l < 2:
      xw = _dense(partn, cnt_src, sc, sh, ws_next[l])
    else:
      out = _final(partn, cnt_src, sc, sh, Wc1, bc1.reshape(1, -1),
                   Wc2, bc2.reshape(1, -1))
  return out


# speedup vs baseline: 3.0085x; 2.4757x over previous
import functools

import jax
import jax.numpy as jnp
from jax import lax
from jax.experimental import pallas as pl
from jax.experimental.pallas import tpu as pltpu
import jax.experimental.pallas.tpu_sc as plsc

N_ = 10000
M_ = 10000
NNZ_ = 320000
F = 64
EPS_ = 1e-5

NC = 2
NS = 16
NW = NC * NS
CH = 80
NCHUNK = 125
MT = M_
RPT = 624
REM = M_ - NS * RPT

_mesh = plsc.VectorSubcoreMesh(core_axis_name="c", subcore_axis_name="s")
_sc_params = pltpu.CompilerParams(use_tc_tiling_on_sc=False)


NBUF = 5
KPF = 3


def _make_scatter(with_counts):
  outs = jax.ShapeDtypeStruct((NC, M_, F), jnp.float32)
  if with_counts:
    outs = (outs,
            jax.ShapeDtypeStruct((NC, N_, 8), jnp.float32),
            jax.ShapeDtypeStruct((NC, M_, 8), jnp.float32))
  scratch = [
      pltpu.VMEM((NCHUNK, CH), jnp.int32),
      pltpu.VMEM((NCHUNK, CH), jnp.int32),
      pltpu.VMEM((NBUF, CH, F), jnp.float32),
      pltpu.VMEM_SHARED((MT, F), jnp.float32),
      [pltpu.SemaphoreType.DMA] * NBUF,
      [pltpu.SemaphoreType.DMA] * NBUF,
  ]
  if with_counts:
    scratch += [
        pltpu.VMEM((NCHUNK, CH), jnp.int32),
        pltpu.VMEM((CH, 8), jnp.float32),
        pltpu.VMEM_SHARED((N_ + 1, 8), jnp.float32),
        pltpu.VMEM_SHARED((M_ + 1, 8), jnp.float32),
        [pltpu.SemaphoreType.DMA] * NBUF,
        [pltpu.SemaphoreType.DMA] * NBUF,
    ]

  @functools.partial(
      pl.kernel,
      out_type=outs,
      mesh=_mesh,
      compiler_params=_sc_params,
      scratch_types=scratch,
  )
  def scatter_kernel(table_hbm, gidx_hbm, sidx_hbm, zeros_hbm, *rest):
    if with_counts:
      (cidx_hbm, ones_hbm, zer8_hbm, out_hbm, cs_out, ch_out,
       gidx_v, sidx_v, rows_v, acc_sh, gsems, ssems,
       cidx_v, ones_v, cs_sh, ch_sh, csems, hsems) = rest
    else:
      (out_hbm, gidx_v, sidx_v, rows_v, acc_sh, gsems, ssems) = rest
    c = lax.axis_index("c")
    s = lax.axis_index("s")
    wid = c * NS + s
    row0 = s * RPT
    pltpu.sync_copy(zeros_hbm.at[pl.ds(row0, RPT)], acc_sh.at[pl.ds(row0, RPT)])

    @pl.when(s == NS - 1)
    def _():
      pltpu.sync_copy(zeros_hbm.at[pl.ds(NS * RPT, REM)],
                      acc_sh.at[pl.ds(NS * RPT, REM)])

    pltpu.sync_copy(gidx_hbm.at[wid], gidx_v)
    pltpu.sync_copy(sidx_hbm.at[wid], sidx_v)
    if with_counts:
      pltpu.sync_copy(cidx_hbm.at[wid], cidx_v)
      pltpu.sync_copy(ones_hbm, ones_v)
      pltpu.sync_copy(zer8_hbm.at[pl.ds(row0, RPT)],
                      cs_sh.at[pl.ds(row0, RPT)])
      pltpu.sync_copy(zer8_hbm.at[pl.ds(row0, RPT)],
                      ch_sh.at[pl.ds(row0, RPT)])

      @pl.when(s == NS - 1)
      def _():
        pltpu.sync_copy(zer8_hbm.at[pl.ds(NS * RPT, REM)],
                        cs_sh.at[pl.ds(NS * RPT, REM)])
        pltpu.sync_copy(zer8_hbm.at[pl.ds(NS * RPT, REM)],
                        ch_sh.at[pl.ds(NS * RPT, REM)])

    plsc.subcore_barrier()

    def fire_gather(i, b):
      pltpu.async_copy(table_hbm.at[gidx_v.at[i]], rows_v.at[b], gsems[b])

    def wait_gather(b):
      pltpu.make_async_copy(table_hbm.at[gidx_v.at[0]], rows_v.at[b],
                            gsems[b]).wait()

    def fire_scatter(i, b):
      pltpu.async_copy(rows_v.at[b], acc_sh.at[sidx_v.at[i]], ssems[b],
                       add=True)

    def wait_scatter(b):
      pltpu.make_async_copy(rows_v.at[b], acc_sh.at[sidx_v.at[0]],
                            ssems[b]).wait()

    def fire_counts(i, b):
      pltpu.async_copy(ones_v, cs_sh.at[cidx_v.at[i]], csems[b], add=True)
      pltpu.async_copy(ones_v, ch_sh.at[sidx_v.at[i]], hsems[b], add=True)

    def drain_counts(b):
      pltpu.make_async_copy(ones_v, cs_sh.at[cidx_v.at[0]], csems[b]).wait()
      pltpu.make_async_copy(ones_v, ch_sh.at[sidx_v.at[0]], hsems[b]).wait()

    for b in range(KPF):
      fire_gather(b, b)

    def outer(i0, carry):
      for b in range(NBUF):
        i = i0 * NBUF + b
        wait_gather(b)
        fire_scatter(i, b)
        if with_counts:

          @pl.when(i >= NBUF)
          def _():
            drain_counts(b)

          fire_counts(i, b)
        pfs = (b + KPF) % NBUF

        @pl.when(i >= NBUF - KPF)
        def _():
          wait_scatter(pfs)

        @pl.when(i + KPF < NCHUNK)
        def _():
          fire_gather(i + KPF, pfs)

      return carry

    lax.fori_loop(0, NCHUNK // NBUF, outer, 0)
    for j in range(NBUF - KPF):
      wait_scatter((NCHUNK - 1 - j) % NBUF)
    if with_counts:
      for b in range(NBUF):
        drain_counts(b)
    plsc.subcore_barrier()
    pltpu.sync_copy(acc_sh.at[pl.ds(row0, RPT)], out_hbm.at[c, pl.ds(row0, RPT)])
    if with_counts:
      pltpu.sync_copy(cs_sh.at[pl.ds(row0, RPT)], cs_out.at[c, pl.ds(row0, RPT)])
      pltpu.sync_copy(ch_sh.at[pl.ds(row0, RPT)], ch_out.at[c, pl.ds(row0, RPT)])

    @pl.when(s == NS - 1)
    def _():
      pltpu.sync_copy(acc_sh.at[pl.ds(NS * RPT, REM)],
                      out_hbm.at[c, pl.ds(NS * RPT, REM)])
      if with_counts:
        pltpu.sync_copy(cs_sh.at[pl.ds(NS * RPT, REM)],
                        cs_out.at[c, pl.ds(NS * RPT, REM)])
        pltpu.sync_copy(ch_sh.at[pl.ds(NS * RPT, REM)],
                        ch_out.at[c, pl.ds(NS * RPT, REM)])

  return scatter_kernel


def _make_count():
  @functools.partial(
      pl.kernel,
      out_type=(jax.ShapeDtypeStruct((NC, N_, 8), jnp.float32),
                jax.ShapeDtypeStruct((NC, M_, 8), jnp.float32)),
      mesh=_mesh,
      compiler_params=_sc_params,
      scratch_types=[
          pltpu.VMEM((NCHUNK, CH), jnp.int32),
          pltpu.VMEM((NCHUNK, CH), jnp.int32),
          pltpu.VMEM((CH, 8), jnp.float32),
          pltpu.VMEM_SHARED((N_, 8), jnp.float32),
          pltpu.VMEM_SHARED((M_, 8), jnp.float32),
          [pltpu.SemaphoreType.DMA] * NBUF,
          [pltpu.SemaphoreType.DMA] * NBUF,
      ],
  )
  def count_kernel(src_hbm, he_hbm, ones_hbm, zer_hbm, cs_out, ch_out,
                   idxs_v, idxh_v, ones_v, cs_sh, ch_sh, ssems, hsems):
    c = lax.axis_index("c")
    s = lax.axis_index("s")
    wid = c * NS + s
    row0 = s * RPT
    pltpu.sync_copy(zer_hbm.at[pl.ds(row0, RPT)], cs_sh.at[pl.ds(row0, RPT)])
    pltpu.sync_copy(zer_hbm.at[pl.ds(row0, RPT)], ch_sh.at[pl.ds(row0, RPT)])

    @pl.when(s == NS - 1)
    def _():
      pltpu.sync_copy(zer_hbm.at[pl.ds(NS * RPT, REM)],
                      cs_sh.at[pl.ds(NS * RPT, REM)])
      pltpu.sync_copy(zer_hbm.at[pl.ds(NS * RPT, REM)],
                      ch_sh.at[pl.ds(NS * RPT, REM)])

    pltpu.sync_copy(ones_hbm, ones_v)
    pltpu.sync_copy(src_hbm.at[wid], idxs_v)
    pltpu.sync_copy(he_hbm.at[wid], idxh_v)
    plsc.subcore_barrier()

    def fire(i, b):
      pltpu.async_copy(ones_v, cs_sh.at[idxs_v.at[i]], ssems[b], add=True)
      pltpu.async_copy(ones_v, ch_sh.at[idxh_v.at[i]], hsems[b], add=True)

    def drain(b):
      pltpu.make_async_copy(ones_v, cs_sh.at[idxs_v.at[0]], ssems[b]).wait()
      pltpu.make_async_copy(ones_v, ch_sh.at[idxh_v.at[0]], hsems[b]).wait()

    def outer(i0, carry):
      for b in range(NBUF):
        i = i0 * NBUF + b

        @pl.when(i >= NBUF)
        def _():
          drain(b)

        fire(i, b)
      return carry

    lax.fori_loop(0, NCHUNK // NBUF, outer, 0)
    for b in range(NBUF):
      drain(b)
    plsc.subcore_barrier()
    pltpu.sync_copy(cs_sh.at[pl.ds(row0, RPT)], cs_out.at[c, pl.ds(row0, RPT)])
    pltpu.sync_copy(ch_sh.at[pl.ds(row0, RPT)], ch_out.at[c, pl.ds(row0, RPT)])

    @pl.when(s == NS - 1)
    def _():
      pltpu.sync_copy(cs_sh.at[pl.ds(NS * RPT, REM)],
                      cs_out.at[c, pl.ds(NS * RPT, REM)])
      pltpu.sync_copy(ch_sh.at[pl.ds(NS * RPT, REM)],
                      ch_out.at[c, pl.ds(NS * RPT, REM)])

  return count_kernel


_scatter = _make_scatter(False)
_scatter_cnt = _make_scatter(True)


_BR = 1000


def _mm0(x, w):
  def body(x_ref, w_ref, o_ref):
    o_ref[...] = jnp.dot(x_ref[...], w_ref[...],
                         preferred_element_type=jnp.float32)

  return pl.pallas_call(
      body,
      grid=(N_ // _BR,),
      in_specs=[pl.BlockSpec((_BR, 128), lambda i: (i, 0)),
                pl.BlockSpec((128, F), lambda i: (0, 0))],
      out_specs=pl.BlockSpec((_BR, F), lambda i: (i, 0)),
      out_shape=jax.ShapeDtypeStruct((N_, F), jnp.float32),
  )(x, w)


def _combine_e(parte, cnt_he):
  def body(p_ref, c_ref, o_ref):
    cnt = c_ref[0][:, 0:1] + c_ref[1][:, 0:1]
    binv = jnp.where(cnt > 0, 1.0 / cnt, 0.0)
    o_ref[...] = (p_ref[0] + p_ref[1]) * binv

  return pl.pallas_call(
      body,
      grid=(M_ // _BR,),
      in_specs=[pl.BlockSpec((NC, _BR, F), lambda i: (0, i, 0)),
                pl.BlockSpec((NC, _BR, 8), lambda i: (0, i, 0))],
      out_specs=pl.BlockSpec((_BR, F), lambda i: (i, 0)),
      out_shape=jax.ShapeDtypeStruct((M_, F), jnp.float32),
  )(parte, cnt_he)


def _dense(partn, cnt_src, scale, shift, w):
  def body(p_ref, c_ref, sc_ref, sh_ref, w_ref, o_ref):
    cnt = c_ref[0][:, 0:1] + c_ref[1][:, 0:1]
    dinv = jnp.where(cnt > 0, 1.0 / cnt, 0.0)
    h = (p_ref[0] + p_ref[1]) * dinv
    h = jnp.maximum(h * sc_ref[...] + sh_ref[...], 0.0)
    o_ref[...] = jnp.dot(h, w_ref[...], preferred_element_type=jnp.float32)

  return pl.pallas_call(
      body,
      grid=(N_ // _BR,),
      in_specs=[pl.BlockSpec((NC, _BR, F), lambda i: (0, i, 0)),
                pl.BlockSpec((NC, _BR, 8), lambda i: (0, i, 0)),
                pl.BlockSpec((1, F), lambda i: (0, 0)),
                pl.BlockSpec((1, F), lambda i: (0, 0)),
                pl.BlockSpec((F, F), lambda i: (0, 0))],
      out_specs=pl.BlockSpec((_BR, F), lambda i: (i, 0)),
      out_shape=jax.ShapeDtypeStruct((N_, F), jnp.float32),
  )(partn, cnt_src, scale, shift, w)


def _final(partn, cnt_src, scale, shift, wc1, bc1, wc2, bc2):
  def body(p_ref, c_ref, sc_ref, sh_ref, w1_ref, b1_ref, w2_ref, b2_ref,
           o_ref):
    cnt = c_ref[0][:, 0:1] + c_ref[1][:, 0:1]
    dinv = jnp.where(cnt > 0, 1.0 / cnt, 0.0)
    h = (p_ref[0] + p_ref[1]) * dinv
    h = jnp.maximum(h * sc_ref[...] + sh_ref[...], 0.0)
    t = jnp.dot(h, w1_ref[...], preferred_element_type=jnp.float32)
    t = jnp.maximum(t + b1_ref[...], 0.0)
    o_ref[...] = jnp.dot(t, w2_ref[...],
                         preferred_element_type=jnp.float32) + b2_ref[...]

  H2 = F // 2
  NCLS = 10
  return pl.pallas_call(
      body,
      grid=(N_ // _BR,),
      in_specs=[pl.BlockSpec((NC, _BR, F), lambda i: (0, i, 0)),
                pl.BlockSpec((NC, _BR, 8), lambda i: (0, i, 0)),
                pl.BlockSpec((1, F), lambda i: (0, 0)),
                pl.BlockSpec((1, F), lambda i: (0, 0)),
                pl.BlockSpec((F, H2), lambda i: (0, 0)),
                pl.BlockSpec((1, H2), lambda i: (0, 0)),
                pl.BlockSpec((H2, NCLS), lambda i: (0, 0)),
                pl.BlockSpec((1, NCLS), lambda i: (0, 0))],
      out_specs=pl.BlockSpec((_BR, NCLS), lambda i: (i, 0)),
      out_shape=jax.ShapeDtypeStruct((N_, NCLS), jnp.float32),
  )(partn, cnt_src, scale, shift, wc1, bc1, wc2, bc2)


def _affine(b, g, be, rm, rv):
  sc = g * lax.rsqrt(rv + EPS_)
  sh = (b - rm) * sc + be
  return sc.reshape(1, F), sh.reshape(1, F)


def kernel(x, hyperedge_index, W0, b0, g0, be0, rm0, rv0,
           W1, b1, g1, be1, rm1, rv1, W2, b2, g2, be2, rm2, rv2,
           Wc1, bc1, Wc2, bc2):
  shp = (NW, NCHUNK, CH)
  src_g = hyperedge_index[0].reshape(shp)
  he_g = hyperedge_index[1].reshape(shp)
  src_t = src_g
  he_t = he_g

  zeros_mf = jnp.zeros((M_, F), jnp.float32)
  zeros_8 = jnp.zeros((M_, 8), jnp.float32)
  ones_8 = jnp.ones((CH, 8), jnp.float32)

  affines = [_affine(b0, g0, be0, rm0, rv0),
             _affine(b1, g1, be1, rm1, rv1),
             _affine(b2, g2, be2, rm2, rv2)]
  ws_next = [W1, W2]

  xw = _mm0(x, W0)
  for l in range(3):
    if l == 0:
      parte, cnt_src, cnt_he = _scatter_cnt(xw, src_g, he_t, zeros_mf,
                                            src_t, ones_8, zeros_8)
    else:
      parte = _scatter(xw, src_g, he_t, zeros_mf)
    oute = _combine_e(parte, cnt_he)
    partn = _scatter(oute, he_g, src_t, zeros_mf)
    sc, sh = affines[l]
    if l < 2:
      xw = _dense(partn, cnt_src, sc, sh, ws_next[l])
    else:
      out = _final(partn, cnt_src, sc, sh, Wc1, bc1.reshape(1, -1),
                   Wc2, bc2.reshape(1, -1))
  return out


# speedup vs baseline: 3.1717x; 1.0542x over previous
import functools

import jax
import jax.numpy as jnp
from jax import lax
from jax.experimental import pallas as pl
from jax.experimental.pallas import tpu as pltpu
import jax.experimental.pallas.tpu_sc as plsc

N_ = 10000
M_ = 10000
NNZ_ = 320000
F = 64
EPS_ = 1e-5

NC = 2
NS = 16
NW = NC * NS
CH = 80
NCHUNK = 125
MT = M_
RPT = 624
REM = M_ - NS * RPT

_mesh = plsc.VectorSubcoreMesh(core_axis_name="c", subcore_axis_name="s")
_sc_params = pltpu.CompilerParams(use_tc_tiling_on_sc=False)


NBUF = 5
KPF = 3


def _make_scatter(with_counts, nbuf=NBUF, kpf=KPF):
  outs = jax.ShapeDtypeStruct((NC, M_, F), jnp.float32)
  if with_counts:
    outs = (outs,
            jax.ShapeDtypeStruct((NC, N_, 8), jnp.float32),
            jax.ShapeDtypeStruct((NC, M_, 8), jnp.float32))
  scratch = [
      pltpu.VMEM((NCHUNK, CH), jnp.int32),
      pltpu.VMEM((NCHUNK, CH), jnp.int32),
      pltpu.VMEM((nbuf, CH, F), jnp.float32),
      pltpu.VMEM_SHARED((MT, F), jnp.float32),
      [pltpu.SemaphoreType.DMA] * nbuf,
      [pltpu.SemaphoreType.DMA] * nbuf,
  ]
  if with_counts:
    scratch += [
        pltpu.VMEM((NCHUNK, CH), jnp.int32),
        pltpu.VMEM((CH, 8), jnp.float32),
        pltpu.VMEM_SHARED((N_, 8), jnp.float32),
        pltpu.VMEM_SHARED((M_, 8), jnp.float32),
        [pltpu.SemaphoreType.DMA] * nbuf,
        [pltpu.SemaphoreType.DMA] * nbuf,
    ]

  @functools.partial(
      pl.kernel,
      out_type=outs,
      mesh=_mesh,
      compiler_params=_sc_params,
      scratch_types=scratch,
  )
  def scatter_kernel(table_hbm, gidx_hbm, sidx_hbm, zeros_hbm, *rest):
    if with_counts:
      (cidx_hbm, ones_hbm, zer8_hbm, out_hbm, cs_out, ch_out,
       gidx_v, sidx_v, rows_v, acc_sh, gsems, ssems,
       cidx_v, ones_v, cs_sh, ch_sh, csems, hsems) = rest
    else:
      (out_hbm, gidx_v, sidx_v, rows_v, acc_sh, gsems, ssems) = rest
    c = lax.axis_index("c")
    s = lax.axis_index("s")
    wid = c * NS + s
    row0 = s * RPT
    pend = [
        pltpu.async_copy(zeros_hbm.at[pl.ds(row0, RPT)],
                         acc_sh.at[pl.ds(row0, RPT)], gsems[0]),
        pltpu.async_copy(gidx_hbm.at[wid], gidx_v, gsems[1]),
        pltpu.async_copy(sidx_hbm.at[wid], sidx_v, gsems[2]),
    ]
    if with_counts:
      pend += [
          pltpu.async_copy(cidx_hbm.at[wid], cidx_v, gsems[3]),
          pltpu.async_copy(ones_hbm, ones_v, gsems[4]),
          pltpu.async_copy(zer8_hbm.at[pl.ds(row0, RPT)],
                           cs_sh.at[pl.ds(row0, RPT)], csems[0]),
          pltpu.async_copy(zer8_hbm.at[pl.ds(row0, RPT)],
                           ch_sh.at[pl.ds(row0, RPT)], csems[1]),
      ]

    @pl.when(s == NS - 1)
    def _():
      pltpu.sync_copy(zeros_hbm.at[pl.ds(NS * RPT, REM)],
                      acc_sh.at[pl.ds(NS * RPT, REM)])
      if with_counts:
        pltpu.sync_copy(zer8_hbm.at[pl.ds(NS * RPT, REM)],
                        cs_sh.at[pl.ds(NS * RPT, REM)])
        pltpu.sync_copy(zer8_hbm.at[pl.ds(NS * RPT, REM)],
                        ch_sh.at[pl.ds(NS * RPT, REM)])

    for d in pend:
      d.wait()
    plsc.subcore_barrier()

    def fire_gather(i, b):
      pltpu.async_copy(table_hbm.at[gidx_v.at[i]], rows_v.at[b], gsems[b])

    def wait_gather(b):
      pltpu.make_async_copy(table_hbm.at[gidx_v.at[0]], rows_v.at[b],
                            gsems[b]).wait()

    def fire_scatter(i, b):
      pltpu.async_copy(rows_v.at[b], acc_sh.at[sidx_v.at[i]], ssems[b],
                       add=True)

    def wait_scatter(b):
      pltpu.make_async_copy(rows_v.at[b], acc_sh.at[sidx_v.at[0]],
                            ssems[b]).wait()

    def fire_counts(i, b):
      pltpu.async_copy(ones_v, cs_sh.at[cidx_v.at[i]], csems[b], add=True)
      pltpu.async_copy(ones_v, ch_sh.at[sidx_v.at[i]], hsems[b], add=True)

    def drain_counts(b):
      pltpu.make_async_copy(ones_v, cs_sh.at[cidx_v.at[0]], csems[b]).wait()
      pltpu.make_async_copy(ones_v, ch_sh.at[sidx_v.at[0]], hsems[b]).wait()

    for b in range(kpf):
      fire_gather(b, b)

    def step(i, b, tail):
      wait_gather(b)
      fire_scatter(i, b)
      if with_counts:
        if tail:
          drain_counts(b)
        else:

          @pl.when(i >= nbuf)
          def _():
            drain_counts(b)

        fire_counts(i, b)
      pfs = (b + kpf) % nbuf
      if tail:
        wait_scatter(pfs)
      else:

        @pl.when(i >= nbuf - kpf)
        def _():
          wait_scatter(pfs)

        @pl.when(i + kpf < NCHUNK)
        def _():
          fire_gather(i + kpf, pfs)

    full = NCHUNK // nbuf
    tail = NCHUNK % nbuf

    def outer(i0, carry):
      for b in range(nbuf):
        step(i0 * nbuf + b, b, False)
      return carry

    lax.fori_loop(0, full, outer, 0)
    for b in range(tail):
      step(full * nbuf + b, b, True)
    for j in range(nbuf - kpf):
      wait_scatter((NCHUNK - 1 - j) % nbuf)
    if with_counts:
      for b in range(tail, nbuf):
        drain_counts(b)
    plsc.subcore_barrier()
    pltpu.sync_copy(acc_sh.at[pl.ds(row0, RPT)], out_hbm.at[c, pl.ds(row0, RPT)])
    if with_counts:
      pltpu.sync_copy(cs_sh.at[pl.ds(row0, RPT)], cs_out.at[c, pl.ds(row0, RPT)])
      pltpu.sync_copy(ch_sh.at[pl.ds(row0, RPT)], ch_out.at[c, pl.ds(row0, RPT)])

    @pl.when(s == NS - 1)
    def _():
      pltpu.sync_copy(acc_sh.at[pl.ds(NS * RPT, REM)],
                      out_hbm.at[c, pl.ds(NS * RPT, REM)])
      if with_counts:
        pltpu.sync_copy(cs_sh.at[pl.ds(NS * RPT, REM)],
                        cs_out.at[c, pl.ds(NS * RPT, REM)])
        pltpu.sync_copy(ch_sh.at[pl.ds(NS * RPT, REM)],
                        ch_out.at[c, pl.ds(NS * RPT, REM)])

  return scatter_kernel


def _make_count():
  @functools.partial(
      pl.kernel,
      out_type=(jax.ShapeDtypeStruct((NC, N_, 8), jnp.float32),
                jax.ShapeDtypeStruct((NC, M_, 8), jnp.float32)),
      mesh=_mesh,
      compiler_params=_sc_params,
      scratch_types=[
          pltpu.VMEM((NCHUNK, CH), jnp.int32),
          pltpu.VMEM((NCHUNK, CH), jnp.int32),
          pltpu.VMEM((CH, 8), jnp.float32),
          pltpu.VMEM_SHARED((N_, 8), jnp.float32),
          pltpu.VMEM_SHARED((M_, 8), jnp.float32),
          [pltpu.SemaphoreType.DMA] * NBUF,
          [pltpu.SemaphoreType.DMA] * NBUF,
      ],
  )
  def count_kernel(src_hbm, he_hbm, ones_hbm, zer_hbm, cs_out, ch_out,
                   idxs_v, idxh_v, ones_v, cs_sh, ch_sh, ssems, hsems):
    c = lax.axis_index("c")
    s = lax.axis_index("s")
    wid = c * NS + s
    row0 = s * RPT
    pltpu.sync_copy(zer_hbm.at[pl.ds(row0, RPT)], cs_sh.at[pl.ds(row0, RPT)])
    pltpu.sync_copy(zer_hbm.at[pl.ds(row0, RPT)], ch_sh.at[pl.ds(row0, RPT)])

    @pl.when(s == NS - 1)
    def _():
      pltpu.sync_copy(zer_hbm.at[pl.ds(NS * RPT, REM)],
                      cs_sh.at[pl.ds(NS * RPT, REM)])
      pltpu.sync_copy(zer_hbm.at[pl.ds(NS * RPT, REM)],
                      ch_sh.at[pl.ds(NS * RPT, REM)])

    pltpu.sync_copy(ones_hbm, ones_v)
    pltpu.sync_copy(src_hbm.at[wid], idxs_v)
    pltpu.sync_copy(he_hbm.at[wid], idxh_v)
    plsc.subcore_barrier()

    def fire(i, b):
      pltpu.async_copy(ones_v, cs_sh.at[idxs_v.at[i]], ssems[b], add=True)
      pltpu.async_copy(ones_v, ch_sh.at[idxh_v.at[i]], hsems[b], add=True)

    def drain(b):
      pltpu.make_async_copy(ones_v, cs_sh.at[idxs_v.at[0]], ssems[b]).wait()
      pltpu.make_async_copy(ones_v, ch_sh.at[idxh_v.at[0]], hsems[b]).wait()

    def outer(i0, carry):
      for b in range(NBUF):
        i = i0 * NBUF + b

        @pl.when(i >= NBUF)
        def _():
          drain(b)

        fire(i, b)
      return carry

    lax.fori_loop(0, NCHUNK // NBUF, outer, 0)
    for b in range(NBUF):
      drain(b)
    plsc.subcore_barrier()
    pltpu.sync_copy(cs_sh.at[pl.ds(row0, RPT)], cs_out.at[c, pl.ds(row0, RPT)])
    pltpu.sync_copy(ch_sh.at[pl.ds(row0, RPT)], ch_out.at[c, pl.ds(row0, RPT)])

    @pl.when(s == NS - 1)
    def _():
      pltpu.sync_copy(cs_sh.at[pl.ds(NS * RPT, REM)],
                      cs_out.at[c, pl.ds(NS * RPT, REM)])
      pltpu.sync_copy(ch_sh.at[pl.ds(NS * RPT, REM)],
                      ch_out.at[c, pl.ds(NS * RPT, REM)])

  return count_kernel


_scatter = _make_scatter(False, nbuf=10, kpf=6)
_scatter_cnt = _make_scatter(True)


_BR = 1000


def _mm0(x, w):
  def body(x_ref, w_ref, o_ref):
    o_ref[...] = jnp.dot(x_ref[...], w_ref[...],
                         preferred_element_type=jnp.float32)

  return pl.pallas_call(
      body,
      grid=(N_ // _BR,),
      in_specs=[pl.BlockSpec((_BR, 128), lambda i: (i, 0)),
                pl.BlockSpec((128, F), lambda i: (0, 0))],
      out_specs=pl.BlockSpec((_BR, F), lambda i: (i, 0)),
      out_shape=jax.ShapeDtypeStruct((N_, F), jnp.float32),
  )(x, w)


def _combine_e(parte, cnt_he):
  def body(p_ref, c_ref, o_ref):
    cnt = c_ref[0][:, 0:1] + c_ref[1][:, 0:1]
    binv = jnp.where(cnt > 0, 1.0 / cnt, 0.0)
    o_ref[...] = (p_ref[0] + p_ref[1]) * binv

  return pl.pallas_call(
      body,
      grid=(M_ // _BR,),
      in_specs=[pl.BlockSpec((NC, _BR, F), lambda i: (0, i, 0)),
                pl.BlockSpec((NC, _BR, 8), lambda i: (0, i, 0))],
      out_specs=pl.BlockSpec((_BR, F), lambda i: (i, 0)),
      out_shape=jax.ShapeDtypeStruct((M_, F), jnp.float32),
  )(parte, cnt_he)


def _dense(partn, cnt_src, scale, shift, w):
  def body(p_ref, c_ref, sc_ref, sh_ref, w_ref, o_ref):
    cnt = c_ref[0][:, 0:1] + c_ref[1][:, 0:1]
    dinv = jnp.where(cnt > 0, 1.0 / cnt, 0.0)
    h = (p_ref[0] + p_ref[1]) * dinv
    h = jnp.maximum(h * sc_ref[...] + sh_ref[...], 0.0)
    o_ref[...] = jnp.dot(h, w_ref[...], preferred_element_type=jnp.float32)

  return pl.pallas_call(
      body,
      grid=(N_ // _BR,),
      in_specs=[pl.BlockSpec((NC, _BR, F), lambda i: (0, i, 0)),
                pl.BlockSpec((NC, _BR, 8), lambda i: (0, i, 0)),
                pl.BlockSpec((1, F), lambda i: (0, 0)),
                pl.BlockSpec((1, F), lambda i: (0, 0)),
                pl.BlockSpec((F, F), lambda i: (0, 0))],
      out_specs=pl.BlockSpec((_BR, F), lambda i: (i, 0)),
      out_shape=jax.ShapeDtypeStruct((N_, F), jnp.float32),
  )(partn, cnt_src, scale, shift, w)


def _final(partn, cnt_src, scale, shift, wc1, bc1, wc2, bc2):
  def body(p_ref, c_ref, sc_ref, sh_ref, w1_ref, b1_ref, w2_ref, b2_ref,
           o_ref):
    cnt = c_ref[0][:, 0:1] + c_ref[1][:, 0:1]
    dinv = jnp.where(cnt > 0, 1.0 / cnt, 0.0)
    h = (p_ref[0] + p_ref[1]) * dinv
    h = jnp.maximum(h * sc_ref[...] + sh_ref[...], 0.0)
    t = jnp.dot(h, w1_ref[...], preferred_element_type=jnp.float32)
    t = jnp.maximum(t + b1_ref[...], 0.0)
    o_ref[...] = jnp.dot(t, w2_ref[...],
                         preferred_element_type=jnp.float32) + b2_ref[...]

  H2 = F // 2
  NCLS = 10
  return pl.pallas_call(
      body,
      grid=(N_ // _BR,),
      in_specs=[pl.BlockSpec((NC, _BR, F), lambda i: (0, i, 0)),
                pl.BlockSpec((NC, _BR, 8), lambda i: (0, i, 0)),
                pl.BlockSpec((1, F), lambda i: (0, 0)),
                pl.BlockSpec((1, F), lambda i: (0, 0)),
                pl.BlockSpec((F, H2), lambda i: (0, 0)),
                pl.BlockSpec((1, H2), lambda i: (0, 0)),
                pl.BlockSpec((H2, NCLS), lambda i: (0, 0)),
                pl.BlockSpec((1, NCLS), lambda i: (0, 0))],
      out_specs=pl.BlockSpec((_BR, NCLS), lambda i: (i, 0)),
      out_shape=jax.ShapeDtypeStruct((N_, NCLS), jnp.float32),
  )(partn, cnt_src, scale, shift, wc1, bc1, wc2, bc2)


def _affine(b, g, be, rm, rv):
  sc = g * lax.rsqrt(rv + EPS_)
  sh = (b - rm) * sc + be
  return sc.reshape(1, F), sh.reshape(1, F)


def kernel(x, hyperedge_index, W0, b0, g0, be0, rm0, rv0,
           W1, b1, g1, be1, rm1, rv1, W2, b2, g2, be2, rm2, rv2,
           Wc1, bc1, Wc2, bc2):
  shp = (NW, NCHUNK, CH)
  src_g = hyperedge_index[0].reshape(shp)
  he_g = hyperedge_index[1].reshape(shp)
  src_t = src_g
  he_t = he_g

  zeros_mf = jnp.zeros((M_, F), jnp.float32)
  zeros_8 = jnp.zeros((M_, 8), jnp.float32)
  ones_8 = jnp.ones((CH, 8), jnp.float32)

  affines = [_affine(b0, g0, be0, rm0, rv0),
             _affine(b1, g1, be1, rm1, rv1),
             _affine(b2, g2, be2, rm2, rv2)]
  ws_next = [W1, W2]

  xw = _mm0(x, W0)
  for l in range(3):
    if l == 0:
      parte, cnt_src, cnt_he = _scatter_cnt(xw, src_g, he_t, zeros_mf,
                                            src_t, ones_8, zeros_8)
    else:
      parte = _scatter(xw, src_g, he_t, zeros_mf)
    oute = _combine_e(parte, cnt_he)
    partn = _scatter(oute, he_g, src_t, zeros_mf)
    sc, sh = affines[l]
    if l < 2:
      xw = _dense(partn, cnt_src, sc, sh, ws_next[l])
    else:
      out = _final(partn, cnt_src, sc, sh, Wc1, bc1.reshape(1, -1),
                   Wc2, bc2.reshape(1, -1))
  return out
